# baseline (speedup 1.0000x reference)
import jax
import jax.numpy as jnp
from jax.experimental import pallas as pl
from jax.experimental.pallas import tpu as pltpu

_TOKENS = 16384
_HIDDEN = 4096
_EXPERTS = 64
_BT = 1024


def _gate_block(thr_ref, hs_ref, sm_ref, rw_ref, scores_ref, k_ref, mask_ref,
                sn_ref):
    @pl.when(pl.program_id(0) == 0)
    def _():
        sm = sm_ref[...]
        cnorm = jnp.sqrt(jnp.sum(sm * sm, axis=0, keepdims=True))
        sn_ref[...] = sm * (1.0 / jnp.maximum(cnorm, 1e-12))

    hs = hs_ref[...]
    rnorm = jnp.sqrt(jnp.sum(hs * hs, axis=1, keepdims=True))
    hn = hs * (1.0 / jnp.maximum(rnorm, 1e-12))
    scores = jax.lax.dot_general(
        hn, sn_ref[...], (((1,), (0,)), ((), ())),
        preferred_element_type=jnp.float32)
    thr = thr_ref[0]
    mask = scores > thr
    maskf = mask.astype(jnp.float32)
    ones_ee = jnp.ones((_EXPERTS, _EXPERTS), jnp.float32)
    kf = jax.lax.dot_general(
        maskf, jnp.ones((_EXPERTS, 1), jnp.float32),
        (((1,), (0,)), ((), ())), preferred_element_type=jnp.float32)
    k_ref[...] = kf.astype(jnp.int32)
    e = jnp.where(mask, jnp.exp(scores), 0.0)
    s = jax.lax.dot_general(
        e, ones_ee, (((1,), (0,)), ((), ())),
        preferred_element_type=jnp.float32)
    rw = e / jnp.maximum(s, 1e-30)
    rw_ref[...] = jnp.where(s == 0.0, 1.0 / _EXPERTS, rw)
    scores_ref[...] = scores
    mask_ref[...] = mask


def kernel(hidden_states, sim_matrix, threshold):
    grid = (_TOKENS // _BT,)
    out = pl.pallas_call(
        _gate_block,
        grid=grid,
        in_specs=[
            pl.BlockSpec(memory_space=pltpu.SMEM),
            pl.BlockSpec((_BT, _HIDDEN), lambda i: (i, 0)),
            pl.BlockSpec((_HIDDEN, _EXPERTS), lambda i: (0, 0)),
        ],
        out_specs=[
            pl.BlockSpec((_BT, _EXPERTS), lambda i: (i, 0)),
            pl.BlockSpec((_BT, _EXPERTS), lambda i: (i, 0)),
            pl.BlockSpec((_BT, 1), lambda i: (i, 0)),
            pl.BlockSpec((_BT, _EXPERTS), lambda i: (i, 0)),
        ],
        out_shape=[
            jax.ShapeDtypeStruct((_TOKENS, _EXPERTS), jnp.float32),
            jax.ShapeDtypeStruct((_TOKENS, _EXPERTS), jnp.float32),
            jax.ShapeDtypeStruct((_TOKENS, 1), jnp.int32),
            jax.ShapeDtypeStruct((_TOKENS, _EXPERTS), jnp.bool_),
        ],
        scratch_shapes=[pltpu.VMEM((_HIDDEN, _EXPERTS), jnp.float32)],
        compiler_params=pltpu.CompilerParams(
            dimension_semantics=("arbitrary",),
        ),
    )(threshold, hidden_states, sim_matrix)
    routing_weights, scores, k_per_token, activated_mask = out
    return routing_weights, scores, k_per_token.reshape(_TOKENS), activated_mask


# speedup vs baseline: 1.0083x; 1.0083x over previous
import jax
import jax.numpy as jnp
from jax.experimental import pallas as pl
from jax.experimental.pallas import tpu as pltpu

_TOKENS = 16384
_HIDDEN = 4096
_EXPERTS = 64
_BT = 1024
_HK = _HIDDEN // 2


def _gate_block(thr_ref, hsa_ref, hsb_ref, sm_ref, rw_ref, scores_ref, k_ref,
                mask_ref, sn_ref):
    @pl.when(pl.program_id(0) == 0)
    def _():
        sm = sm_ref[...]
        cnorm = jnp.sqrt(jnp.sum(sm * sm, axis=0, keepdims=True))
        sn_ref[...] = sm * (1.0 / jnp.maximum(cnorm, 1e-12))

    hsa = hsa_ref[...]
    hsb = hsb_ref[...]
    ssq = (jnp.sum(hsa * hsa, axis=1, keepdims=True) +
           jnp.sum(hsb * hsb, axis=1, keepdims=True))
    inv = 1.0 / jnp.maximum(jnp.sqrt(ssq), 1e-12)
    dn = (((1,), (0,)), ((), ()))
    scores = (jax.lax.dot_general(hsa * inv, sn_ref[:_HK, :], dn,
                                  preferred_element_type=jnp.float32) +
              jax.lax.dot_general(hsb * inv, sn_ref[_HK:, :], dn,
                                  preferred_element_type=jnp.float32))
    thr = thr_ref[0]
    mask = scores > thr
    maskf = mask.astype(jnp.float32)
    ones_ee = jnp.ones((_EXPERTS, _EXPERTS), jnp.float32)
    kf = jax.lax.dot_general(
        maskf, jnp.ones((_EXPERTS, 1), jnp.float32),
        (((1,), (0,)), ((), ())), preferred_element_type=jnp.float32)
    k_ref[...] = kf.astype(jnp.int32)
    e = jnp.where(mask, jnp.exp(scores), 0.0)
    s = jax.lax.dot_general(
        e, ones_ee, (((1,), (0,)), ((), ())),
        preferred_element_type=jnp.float32)
    rw = e / jnp.maximum(s, 1e-30)
    rw_ref[...] = jnp.where(s == 0.0, 1.0 / _EXPERTS, rw)
    scores_ref[...] = scores
    mask_ref[...] = mask


def kernel(hidden_states, sim_matrix, threshold):
    grid = (_TOKENS // _BT,)
    out = pl.pallas_call(
        _gate_block,
        grid=grid,
        in_specs=[
            pl.BlockSpec(memory_space=pltpu.SMEM),
            pl.BlockSpec((_BT, _HK), lambda i: (i, 0)),
            pl.BlockSpec((_BT, _HK), lambda i: (i, 1)),
            pl.BlockSpec((_HIDDEN, _EXPERTS), lambda i: (0, 0)),
        ],
        out_specs=[
            pl.BlockSpec((_BT, _EXPERTS), lambda i: (i, 0)),
            pl.BlockSpec((_BT, _EXPERTS), lambda i: (i, 0)),
            pl.BlockSpec((_BT, 1), lambda i: (i, 0)),
            pl.BlockSpec((_BT, _EXPERTS), lambda i: (i, 0)),
        ],
        out_shape=[
            jax.ShapeDtypeStruct((_TOKENS, _EXPERTS), jnp.float32),
            jax.ShapeDtypeStruct((_TOKENS, _EXPERTS), jnp.float32),
            jax.ShapeDtypeStruct((_TOKENS, 1), jnp.int32),
            jax.ShapeDtypeStruct((_TOKENS, _EXPERTS), jnp.bool_),
        ],
        scratch_shapes=[pltpu.VMEM((_HIDDEN, _EXPERTS), jnp.float32)],
        compiler_params=pltpu.CompilerParams(
            dimension_semantics=("arbitrary",),
        ),
    )(threshold, hidden_states, hidden_states, sim_matrix)
    routing_weights, scores, k_per_token, activated_mask = out
    return routing_weights, scores, k_per_token.reshape(_TOKENS), activated_mask


# speedup vs baseline: 1.0151x; 1.0067x over previous
import jax
import jax.numpy as jnp
from jax.experimental import pallas as pl
from jax.experimental.pallas import tpu as pltpu

_TOKENS = 16384
_HIDDEN = 4096
_EXPERTS = 64
_BT = 512
_NC = _TOKENS // _BT
_DEPTH = 4


def _start_chunk_copy(hs_hbm, bufs, sems, c):
    slot = jax.lax.rem(c, _DEPTH)
    pltpu.make_async_copy(
        hs_hbm.at[pl.ds(c * _BT, _BT), :],
        bufs.at[slot],
        sems.at[slot],
    ).start()


def _gate_block(thr_ref, hs_hbm, sm_ref, rw_ref, scores_ref, k_ref,
                mask_ref, sn_ref, bufs, sems):
    i = pl.program_id(0)

    @pl.when(i == 0)
    def _():
        sm = sm_ref[...]
        cnorm = jnp.sqrt(jnp.sum(sm * sm, axis=0, keepdims=True))
        sn_ref[...] = sm * (1.0 / jnp.maximum(cnorm, 1e-12))
        _start_chunk_copy(hs_hbm, bufs, sems, 0)
        _start_chunk_copy(hs_hbm, bufs, sems, 1)

    @pl.when(i + 2 < _NC)
    def _():
        _start_chunk_copy(hs_hbm, bufs, sems, i + 2)

    slot = jax.lax.rem(i, _DEPTH)
    pltpu.make_async_copy(
        hs_hbm.at[pl.ds(i * _BT, _BT), :], bufs.at[slot], sems.at[slot]
    ).wait()

    hs = bufs[slot]
    rnorm = jnp.sqrt(jnp.sum(hs * hs, axis=1, keepdims=True))
    hn = hs * (1.0 / jnp.maximum(rnorm, 1e-12))
    scores = jax.lax.dot_general(
        hn, sn_ref[...], (((1,), (0,)), ((), ())),
        preferred_element_type=jnp.float32)
    thr = thr_ref[0]
    mask = scores > thr
    maskf = mask.astype(jnp.float32)
    ones_ee = jnp.ones((_EXPERTS, _EXPERTS), jnp.float32)
    kf = jax.lax.dot_general(
        maskf, jnp.ones((_EXPERTS, 1), jnp.float32),
        (((1,), (0,)), ((), ())), preferred_element_type=jnp.float32)
    k_ref[...] = kf.astype(jnp.int32)
    e = jnp.where(mask, jnp.exp(scores), 0.0)
    s = jax.lax.dot_general(
        e, ones_ee, (((1,), (0,)), ((), ())),
        preferred_element_type=jnp.float32)
    rw = e / jnp.maximum(s, 1e-30)
    rw_ref[...] = jnp.where(s == 0.0, 1.0 / _EXPERTS, rw)
    scores_ref[...] = scores
    mask_ref[...] = mask


def kernel(hidden_states, sim_matrix, threshold):
    grid = (_NC,)
    out = pl.pallas_call(
        _gate_block,
        grid=grid,
        in_specs=[
            pl.BlockSpec(memory_space=pltpu.SMEM),
            pl.BlockSpec(memory_space=pl.ANY),
            pl.BlockSpec((_HIDDEN, _EXPERTS), lambda i: (0, 0)),
        ],
        out_specs=[
            pl.BlockSpec((_BT, _EXPERTS), lambda i: (i, 0)),
            pl.BlockSpec((_BT, _EXPERTS), lambda i: (i, 0)),
            pl.BlockSpec((_BT, 1), lambda i: (i, 0)),
            pl.BlockSpec((_BT, _EXPERTS), lambda i: (i, 0)),
        ],
        out_shape=[
            jax.ShapeDtypeStruct((_TOKENS, _EXPERTS), jnp.float32),
            jax.ShapeDtypeStruct((_TOKENS, _EXPERTS), jnp.float32),
            jax.ShapeDtypeStruct((_TOKENS, 1), jnp.int32),
            jax.ShapeDtypeStruct((_TOKENS, _EXPERTS), jnp.bool_),
        ],
        scratch_shapes=[
            pltpu.VMEM((_HIDDEN, _EXPERTS), jnp.float32),
            pltpu.VMEM((_DEPTH, _BT, _HIDDEN), jnp.float32),
            pltpu.SemaphoreType.DMA((_DEPTH,)),
        ],
        compiler_params=pltpu.CompilerParams(
            dimension_semantics=("arbitrary",),
        ),
    )(threshold, hidden_states, sim_matrix)
    routing_weights, scores, k_per_token, activated_mask = out
    return routing_weights, scores, k_per_token.reshape(_TOKENS), activated_mask
